# baseline (speedup 1.0000x reference)
import functools

import jax
import jax.numpy as jnp
from jax import lax
from jax.experimental import pallas as pl
from jax.experimental.pallas import tpu as pltpu
from jax.experimental.pallas import tpu_sc as plsc

_NC = 2
_NS = 16
_NW = _NC * _NS

_CHUNK_ROWS = 32
_SC_ROWS = 2048
_TC_BLOCK_ROWS = 2048


def _sc_copy(rows, d):
    rows_per_w = rows // _NW
    n_chunks = rows_per_w // _CHUNK_ROWS
    mesh = plsc.VectorSubcoreMesh(core_axis_name="c", subcore_axis_name="s")

    @functools.partial(
        pl.kernel,
        out_type=jax.ShapeDtypeStruct((rows, d), jnp.float32),
        mesh=mesh,
        scratch_types=[
            pltpu.VMEM((_CHUNK_ROWS, d), jnp.float32),
            pltpu.SemaphoreType.DMA,
        ],
    )
    def k(x_hbm, o_hbm, buf, sem):
        wid = lax.axis_index("s") * _NC + lax.axis_index("c")
        base = wid * rows_per_w

        def step(i, carry):
            off = base + i * _CHUNK_ROWS
            pltpu.async_copy(x_hbm.at[pl.ds(off, _CHUNK_ROWS)], buf, sem).wait()
            pltpu.async_copy(buf, o_hbm.at[pl.ds(off, _CHUNK_ROWS)], sem).wait()
            return carry

        lax.fori_loop(0, n_chunks, step, 0)

    return k


def _tc_copy_body(x_ref, o_ref):
    o_ref[...] = x_ref[...]


def _tc_copy(rows, d, x):
    grid = (pl.cdiv(rows, _TC_BLOCK_ROWS),)
    return pl.pallas_call(
        _tc_copy_body,
        out_shape=jax.ShapeDtypeStruct((rows, d), jnp.float32),
        grid=grid,
        in_specs=[pl.BlockSpec((_TC_BLOCK_ROWS, d), lambda i: (i, 0))],
        out_specs=pl.BlockSpec((_TC_BLOCK_ROWS, d), lambda i: (i, 0)),
    )(x)


def kernel(x, gate_w, gate_b, w1, b1, w2, b2):
    b, n, d = x.shape
    x2 = x.reshape(b * n, d)
    rows = b * n
    tc_rows = rows - _SC_ROWS
    out_tc = _tc_copy(tc_rows, d, x2[:tc_rows])
    out_sc = _sc_copy(_SC_ROWS, d)(x2[tc_rows:])
    out = jnp.concatenate([out_tc, out_sc], axis=0)
    return out.reshape(b, n, d)


# speedup vs baseline: 1.8971x; 1.8971x over previous
import functools

import jax
import jax.numpy as jnp
from jax import lax
from jax.experimental import pallas as pl
from jax.experimental.pallas import tpu as pltpu
from jax.experimental.pallas import tpu_sc as plsc

_NC = 2
_NS = 16
_NW = _NC * _NS

_CHUNK_ROWS = 32


def _sc_copy(rows, d):
    rows_per_w = rows // _NW
    n_chunks = rows_per_w // _CHUNK_ROWS
    mesh = plsc.VectorSubcoreMesh(core_axis_name="c", subcore_axis_name="s")

    @functools.partial(
        pl.kernel,
        out_type=jax.ShapeDtypeStruct((rows, d), jnp.float32),
        mesh=mesh,
        scratch_types=[
            pltpu.VMEM((2, _CHUNK_ROWS, d), jnp.float32),
            pltpu.SemaphoreType.DMA((2,)),
            pltpu.SemaphoreType.DMA((2,)),
        ],
    )
    def k(x_hbm, o_hbm, buf, rsem, wsem):
        wid = lax.axis_index("s") * _NC + lax.axis_index("c")
        base = wid * rows_per_w

        def sl(c):
            return pl.ds(base + c * _CHUNK_ROWS, _CHUNK_ROWS)

        reads = [None] * n_chunks
        writes = [None] * n_chunks
        reads[0] = pltpu.async_copy(x_hbm.at[sl(0)], buf.at[0], rsem.at[0])
        for c in range(n_chunks):
            b = c % 2
            reads[c].wait()
            writes[c] = pltpu.async_copy(buf.at[b], o_hbm.at[sl(c)], wsem.at[b])
            if c + 1 < n_chunks:
                if c >= 1:
                    writes[c - 1].wait()
                reads[c + 1] = pltpu.async_copy(
                    x_hbm.at[sl(c + 1)], buf.at[1 - b], rsem.at[1 - b]
                )
        if n_chunks >= 2:
            writes[n_chunks - 2].wait()
        writes[n_chunks - 1].wait()

    return k


def kernel(x, gate_w, gate_b, w1, b1, w2, b2):
    b, n, d = x.shape
    x2 = x.reshape(b * n, d)
    out = _sc_copy(b * n, d)(x2)
    return out.reshape(b, n, d)
